# baseline (speedup 1.0000x reference)
import functools

import jax
import jax.numpy as jnp
from jax.experimental import pallas as pl
from jax.experimental.pallas import tpu as pltpu


def _fold_weights(w_oihw):
    w = jnp.transpose(w_oihw, (2, 3, 1, 0)).astype(jnp.float32)
    fold = {0: ((0,), (1, 2)), 1: ((0, 1), (2,))}
    phases = []
    for a in range(2):
        for c in range(2):
            for ky in range(2):
                for kx in range(2):
                    t = 0.0
                    for dy in fold[a][ky]:
                        for dx in fold[c][kx]:
                            t = t + w[dy, dx]
                    phases.append(t)
    cin, cout = phases[0].shape
    return jnp.stack(phases).reshape(2, 2, 4, cin, cout)


def _up_conv_kernel(x_ref, w_ref, b_ref, o_ref, xp_ref):
    bt, H, W, Cin = x_ref.shape
    Cout = w_ref.shape[4]
    M = bt * H * W

    xp_ref[:, 0:1, :, :] = jnp.zeros((bt, 1, W + 2, Cin), jnp.bfloat16)
    xp_ref[:, H + 1:H + 2, :, :] = jnp.zeros((bt, 1, W + 2, Cin), jnp.bfloat16)
    xp_ref[:, 1:H + 1, 0:1, :] = jnp.zeros((bt, H, 1, Cin), jnp.bfloat16)
    xp_ref[:, 1:H + 1, W + 1:W + 2, :] = jnp.zeros((bt, H, 1, Cin), jnp.bfloat16)
    xp_ref[:, 1:H + 1, 1:W + 1, :] = x_ref[...]

    bias2 = b_ref[...]
    win = {}
    for dy in range(3):
        for dx in range(3):
            win[dy, dx] = xp_ref[:, dy:dy + H, dx:dx + W, :].reshape(M, Cin)
    for a in range(2):
        cols = []
        for c in range(2):
            lhs = jnp.concatenate(
                [win[a + ky, c + kx] for ky in range(2) for kx in range(2)],
                axis=1)
            acc = jnp.dot(lhs, w_ref[a, c].reshape(4 * Cin, Cout),
                          preferred_element_type=jnp.float32)
            cols.append(acc)
        row = jnp.concatenate(cols, axis=-1) + bias2
        r4 = row.reshape(bt, H, W, 2 * Cout).astype(jnp.bfloat16)
        for i in range(H):
            o_ref[:, 2 * i + a, :, :] = r4[:, i]


def kernel(x_nchw, w_oihw, bias):
    B, Cin, H, W = x_nchw.shape
    Cout = w_oihw.shape[0]
    bt = next(d for d in (4, 2, 1) if B % d == 0)

    xh = jnp.transpose(x_nchw, (0, 2, 3, 1)).astype(jnp.bfloat16)
    wf = _fold_weights(w_oihw).astype(jnp.bfloat16)
    b2 = jnp.concatenate([bias, bias]).reshape(1, 2 * Cout).astype(jnp.float32)

    flops = 2 * B * 4 * H * W * 4 * Cin * Cout
    bytes_accessed = int(xh.size * 2 + B * H * W * 4 * Cout * 2 + wf.size * 2)

    out = pl.pallas_call(
        _up_conv_kernel,
        out_shape=jax.ShapeDtypeStruct((B, 2 * H, W, 2 * Cout), jnp.bfloat16),
        grid=(B // bt,),
        in_specs=[
            pl.BlockSpec((bt, H, W, Cin), lambda i: (i, 0, 0, 0)),
            pl.BlockSpec((2, 2, 4, Cin, Cout), lambda i: (0, 0, 0, 0, 0)),
            pl.BlockSpec((1, 2 * Cout), lambda i: (0, 0)),
        ],
        out_specs=pl.BlockSpec((bt, 2 * H, W, 2 * Cout),
                               lambda i: (i, 0, 0, 0)),
        scratch_shapes=[pltpu.VMEM((bt, H + 2, W + 2, Cin), jnp.bfloat16)],
        compiler_params=pltpu.CompilerParams(
            dimension_semantics=("parallel",),
            vmem_limit_bytes=56 * 1024 * 1024),
        cost_estimate=pl.CostEstimate(
            flops=flops, transcendentals=0, bytes_accessed=bytes_accessed),
    )(xh, wf, b2)

    y = out.reshape(B, 2 * H, W, 2, Cout).transpose(0, 4, 1, 2, 3)
    return y.astype(jnp.float32).reshape(B, Cout, 2 * H, 2 * W)


# speedup vs baseline: 1.0059x; 1.0059x over previous
import functools

import jax
import jax.numpy as jnp
from jax.experimental import pallas as pl
from jax.experimental.pallas import tpu as pltpu


def _fold_weights(w_oihw):
    w = jnp.transpose(w_oihw, (2, 3, 1, 0)).astype(jnp.float32)
    fold = {0: ((0,), (1, 2)), 1: ((0, 1), (2,))}
    phases = []
    for a in range(2):
        for c in range(2):
            for ky in range(2):
                for kx in range(2):
                    t = 0.0
                    for dy in fold[a][ky]:
                        for dx in fold[c][kx]:
                            t = t + w[dy, dx]
                    phases.append(t)
    cin, cout = phases[0].shape
    return jnp.stack(phases).reshape(2, 2, 4, cin, cout)


def _up_conv_kernel(x_ref, w_ref, b_ref, o_ref, xp_ref):
    bt, H, W, Cin = x_ref.shape
    Cout = w_ref.shape[4]
    M = bt * H * W

    xp_ref[:, 0:1, :, :] = jnp.zeros((bt, 1, W + 2, Cin), jnp.bfloat16)
    xp_ref[:, H + 1:H + 2, :, :] = jnp.zeros((bt, 1, W + 2, Cin), jnp.bfloat16)
    xp_ref[:, 1:H + 1, 0:1, :] = jnp.zeros((bt, H, 1, Cin), jnp.bfloat16)
    xp_ref[:, 1:H + 1, W + 1:W + 2, :] = jnp.zeros((bt, H, 1, Cin), jnp.bfloat16)
    xp_ref[:, 1:H + 1, 1:W + 1, :] = x_ref[...]

    bias2 = b_ref[...]
    win = {}
    for dy in range(3):
        for dx in range(3):
            win[dy, dx] = xp_ref[:, dy:dy + H, dx:dx + W, :].reshape(M, Cin)
    for a in range(2):
        cols = []
        for c in range(2):
            lhs = jnp.concatenate(
                [win[a + ky, c + kx] for ky in range(2) for kx in range(2)],
                axis=1)
            acc = jnp.dot(lhs, w_ref[a, c].reshape(4 * Cin, Cout),
                          preferred_element_type=jnp.float32)
            cols.append(acc)
        row = jnp.concatenate(cols, axis=-1) + bias2
        r4 = row.reshape(bt, H, W, 2 * Cout).astype(jnp.bfloat16)
        for i in range(H):
            o_ref[:, 2 * i + a, :, :] = r4[:, i]


def kernel(x_nchw, w_oihw, bias):
    B, Cin, H, W = x_nchw.shape
    Cout = w_oihw.shape[0]
    bt = next(d for d in (8, 4, 2, 1) if B % d == 0)

    xh = jnp.transpose(x_nchw, (0, 2, 3, 1)).astype(jnp.bfloat16)
    wf = _fold_weights(w_oihw).astype(jnp.bfloat16)
    b2 = jnp.concatenate([bias, bias]).reshape(1, 2 * Cout).astype(jnp.float32)

    flops = 2 * B * 4 * H * W * 4 * Cin * Cout
    bytes_accessed = int(xh.size * 2 + B * H * W * 4 * Cout * 2 + wf.size * 2)

    out = pl.pallas_call(
        _up_conv_kernel,
        out_shape=jax.ShapeDtypeStruct((B, 2 * H, W, 2 * Cout), jnp.bfloat16),
        grid=(B // bt,),
        in_specs=[
            pl.BlockSpec((bt, H, W, Cin), lambda i: (i, 0, 0, 0)),
            pl.BlockSpec((2, 2, 4, Cin, Cout), lambda i: (0, 0, 0, 0, 0)),
            pl.BlockSpec((1, 2 * Cout), lambda i: (0, 0)),
        ],
        out_specs=pl.BlockSpec((bt, 2 * H, W, 2 * Cout),
                               lambda i: (i, 0, 0, 0)),
        scratch_shapes=[pltpu.VMEM((bt, H + 2, W + 2, Cin), jnp.bfloat16)],
        compiler_params=pltpu.CompilerParams(
            dimension_semantics=("parallel",),
            vmem_limit_bytes=56 * 1024 * 1024),
        cost_estimate=pl.CostEstimate(
            flops=flops, transcendentals=0, bytes_accessed=bytes_accessed),
    )(xh, wf, b2)

    y = out.reshape(B, 2 * H, W, 2, Cout).transpose(0, 4, 1, 2, 3)
    return y.astype(jnp.float32).reshape(B, Cout, 2 * H, 2 * W)
